# Initial kernel scaffold; baseline (speedup 1.0000x reference)
import jax
import jax.numpy as jnp
from jax.experimental import pallas as pl


def kernel(x, adj_list_0, adj_list_1, adj_list_2, adj_list_3, W, b):
    raise NotImplementedError("write your pallas kernel here")


# speedup vs baseline: 2.0060x; 2.0060x over previous
import functools

import jax
import jax.numpy as jnp
from jax import lax
from jax.experimental import pallas as pl
from jax.experimental.pallas import tpu as pltpu
from jax.experimental.pallas import tpu_sc as plsc

N = 10000
D = 256
MSG = 256
T = 4
E_PER = 40000

NROWS = 10240
NACC = 10112
E_PAD = 40960
CH = 128
CHUNKS = E_PAD // (16 * CH)
HALF = 128
NB = 2048


def _tables(xp, W, b8):
    nblk = NROWS // NB

    def mm(x_ref, w_ref, b_ref, a_ref, t_ref):
        xa = x_ref[...]
        w = w_ref[0]
        a_ref[0, 0] = jnp.dot(xa, w[:D, :], preferred_element_type=jnp.float32)
        t_ref[0, 0] = (jnp.dot(xa, w[D:, :], preferred_element_type=jnp.float32)
                       + b_ref[0, 0][None, :])

    return pl.pallas_call(
        mm,
        grid=(nblk, 2, T),
        in_specs=[
            pl.BlockSpec((NB, D), lambda nb, c, t: (nb, 0)),
            pl.BlockSpec((1, 2 * D, HALF), lambda nb, c, t: (t, 0, c)),
            pl.BlockSpec((1, 8, HALF), lambda nb, c, t: (t, 0, c)),
        ],
        out_specs=[
            pl.BlockSpec((1, 1, NB, HALF), lambda nb, c, t: (c, t, nb, 0)),
            pl.BlockSpec((1, 1, NB, HALF), lambda nb, c, t: (c, t, nb, 0)),
        ],
        out_shape=[jax.ShapeDtypeStruct((2, T, NROWS, HALF), jnp.float32)] * 2,
    )(xp, W, b8)


def _edge_stage(tabA, tabB, srcs, tgts):
    mesh = plsc.VectorSubcoreMesh(core_axis_name="c", subcore_axis_name="s")

    @functools.partial(
        pl.kernel,
        out_type=jax.ShapeDtypeStruct((2, NACC, HALF), jnp.float32),
        mesh=mesh,
        scratch_types=[
            pltpu.VMEM((CH,), jnp.int32),
            pltpu.VMEM((CH,), jnp.int32),
            pltpu.VMEM((CH, HALF), jnp.float32),
            pltpu.VMEM((CH, HALF), jnp.float32),
            pltpu.VMEM_SHARED((NACC, HALF), jnp.float32),
        ],
    )
    def edge_kernel(tabA_hbm, tabB_hbm, srcs_hbm, tgts_hbm, out_hbm,
                    sidx, tidx, srows, trows, acc):
        c = lax.axis_index("c")
        s = lax.axis_index("s")

        @pl.loop(0, CH)
        def _(i):
            for j in range(HALF // 16):
                srows[i, pl.ds(j * 16, 16)] = jnp.zeros((16,), jnp.float32)

        rows_per_sub = NACC // 16

        @pl.loop(0, 4)
        def _(k):
            pltpu.sync_copy(srows, acc.at[pl.ds(s * rows_per_sub + k * CH, CH)])

        pltpu.sync_copy(srows.at[pl.ds(0, rows_per_sub - 4 * CH)],
                        acc.at[pl.ds(s * rows_per_sub + 4 * CH,
                                     rows_per_sub - 4 * CH)])

        plsc.subcore_barrier()

        for t in range(T):
            @pl.loop(0, CHUNKS)
            def _(k):
                off = s * (CHUNKS * CH) + k * CH
                pltpu.sync_copy(srcs_hbm.at[t, pl.ds(off, CH)], sidx)
                pltpu.sync_copy(tgts_hbm.at[t, pl.ds(off, CH)], tidx)
                pltpu.sync_copy(tabA_hbm.at[c, t].at[sidx], srows)
                pltpu.sync_copy(tabB_hbm.at[c, t].at[tidx], trows)

                @pl.loop(0, CH)
                def _(i):
                    for j in range(HALF // 16):
                        sl = pl.ds(j * 16, 16)
                        srows[i, sl] = jnp.maximum(
                            srows[i, sl] + trows[i, sl], 0.0)

                pltpu.sync_copy(srows, acc.at[tidx], add=True)

        plsc.subcore_barrier()

        r0 = s * rows_per_sub
        pltpu.sync_copy(acc.at[pl.ds(r0, rows_per_sub)],
                        out_hbm.at[c, pl.ds(r0, rows_per_sub)])

    return edge_kernel(tabA, tabB, srcs, tgts)


def kernel(x, adj_list_0, adj_list_1, adj_list_2, adj_list_3, W, b):
    adj = jnp.stack([adj_list_0, adj_list_1, adj_list_2, adj_list_3])
    srcs = jnp.concatenate(
        [adj[:, :, 0], jnp.zeros((T, E_PAD - E_PER), jnp.int32)], axis=1)
    tgts = jnp.concatenate(
        [adj[:, :, 1], jnp.full((T, E_PAD - E_PER), N, jnp.int32)], axis=1)

    xp = jnp.pad(x, ((0, NROWS - N), (0, 0)))
    b8 = jnp.broadcast_to(b[:, None, :], (T, 8, MSG))

    tabA, tabB = _tables(xp, W, b8)
    out2 = _edge_stage(tabA, tabB, srcs, tgts)
    return jnp.concatenate([out2[0, :N], out2[1, :N]], axis=1)


# speedup vs baseline: 3.4771x; 1.7333x over previous
import functools

import jax
import jax.numpy as jnp
from jax import lax
from jax.experimental import pallas as pl
from jax.experimental.pallas import tpu as pltpu
from jax.experimental.pallas import tpu_sc as plsc

N = 10000
D = 256
MSG = 256
T = 4
E_PER = 40000

NROWS = 10240
NACC = 10112
E_PAD = 40960
CH = 40
CHUNKS = E_PAD // (16 * CH)
HALF = 128
NB = 2048


def _tables(xp, W, b8):
    nblk = NROWS // NB

    def mm(x_ref, w_ref, b_ref, a_ref, t_ref):
        xa = x_ref[...]
        w = w_ref[0]
        a_ref[0, 0] = jnp.dot(xa, w[:D, :], preferred_element_type=jnp.float32)
        t_ref[0, 0] = (jnp.dot(xa, w[D:, :], preferred_element_type=jnp.float32)
                       + b_ref[0, 0][None, :])

    return pl.pallas_call(
        mm,
        grid=(nblk, 2, T),
        in_specs=[
            pl.BlockSpec((NB, D), lambda nb, c, t: (nb, 0)),
            pl.BlockSpec((1, 2 * D, HALF), lambda nb, c, t: (t, 0, c)),
            pl.BlockSpec((1, 8, HALF), lambda nb, c, t: (t, 0, c)),
        ],
        out_specs=[
            pl.BlockSpec((1, 1, NB, HALF), lambda nb, c, t: (c, t, nb, 0)),
            pl.BlockSpec((1, 1, NB, HALF), lambda nb, c, t: (c, t, nb, 0)),
        ],
        out_shape=[jax.ShapeDtypeStruct((2, T, NROWS, HALF), jnp.float32)] * 2,
    )(xp, W, b8)


def _edge_stage(tabAf, tabBf, eidx):
    mesh = plsc.VectorSubcoreMesh(core_axis_name="c", subcore_axis_name="s")
    Q = T * CHUNKS

    @functools.partial(
        pl.kernel,
        out_type=jax.ShapeDtypeStruct((2, NACC, HALF), jnp.float32),
        mesh=mesh,
        scratch_types=[
            pltpu.VMEM((4, 3, CH), jnp.int32),
            pltpu.VMEM((CH, HALF), jnp.float32),
            pltpu.VMEM((CH, HALF), jnp.float32),
            pltpu.VMEM((CH, HALF), jnp.float32),
            pltpu.VMEM((CH, HALF), jnp.float32),
            pltpu.VMEM((CH, HALF), jnp.float32),
            pltpu.VMEM((CH, HALF), jnp.float32),
            pltpu.VMEM_SHARED((NACC, HALF), jnp.float32),
            pltpu.SemaphoreType.DMA,
            pltpu.SemaphoreType.DMA,
            pltpu.SemaphoreType.DMA,
            pltpu.SemaphoreType.DMA,
            pltpu.SemaphoreType.DMA,
            pltpu.SemaphoreType.DMA,
            pltpu.SemaphoreType.DMA,
            pltpu.SemaphoreType.DMA,
            pltpu.SemaphoreType.DMA,
            pltpu.SemaphoreType.DMA,
        ],
    )
    def edge_kernel(tabA_hbm, tabB_hbm, eidx_hbm, out_hbm,
                    idxb, sb0, sb1, tb0, tb1, mb0, mb1, acc,
                    semI0, semI1, semI2, semI3,
                    semA0, semA1, semB0, semB1, semS0, semS1):
        c = lax.axis_index("c")
        s = lax.axis_index("s")
        sbuf, tbuf, mbuf = (sb0, sb1), (tb0, tb1), (mb0, mb1)
        semI = (semI0, semI1, semI2, semI3)
        semA, semB, semS = (semA0, semA1), (semB0, semB1), (semS0, semS1)
        tabAc = tabA_hbm.at[c]
        tabBc = tabB_hbm.at[c]
        eidx_s = eidx_hbm.at[s]

        @pl.loop(0, CH)
        def _(i):
            for j in range(HALF // 16):
                mb0[i, pl.ds(j * 16, 16)] = jnp.zeros((16,), jnp.float32)

        rows_per_sub = NACC // 16
        nz, rz = rows_per_sub // CH, rows_per_sub % CH

        @pl.loop(0, nz)
        def _(k):
            pltpu.sync_copy(mb0, acc.at[pl.ds(s * rows_per_sub + k * CH, CH)])

        if rz:
            pltpu.sync_copy(mb0.at[pl.ds(0, rz)],
                            acc.at[pl.ds(s * rows_per_sub + nz * CH, rz)])

        plsc.subcore_barrier()

        def issue_i(q, islot):
            pltpu.async_copy(eidx_s.at[q], idxb.at[islot], semI[islot])

        def wait_i(q, islot):
            pltpu.make_async_copy(
                eidx_s.at[q], idxb.at[islot], semI[islot]).wait()

        def issue_g(q, b, islot):
            pltpu.async_copy(tabAc.at[idxb.at[islot, 0]], sbuf[b], semA[b])
            pltpu.async_copy(tabBc.at[idxb.at[islot, 1]], tbuf[b], semB[b])

        def wait_g(b, islot):
            pltpu.make_async_copy(
                tabAc.at[idxb.at[islot, 0]], sbuf[b], semA[b]).wait()
            pltpu.make_async_copy(
                tabBc.at[idxb.at[islot, 1]], tbuf[b], semB[b]).wait()

        def compute(b):
            sb, tb, mb = sbuf[b], tbuf[b], mbuf[b]

            @pl.loop(0, CH)
            def _(i):
                for j in range(HALF // 16):
                    sl = pl.ds(j * 16, 16)
                    mb[i, sl] = jnp.maximum(sb[i, sl] + tb[i, sl], 0.0)

        def issue_s(b, islot):
            pltpu.async_copy(mbuf[b], acc.at[idxb.at[islot, 2]], semS[b],
                             add=True)

        def wait_s(b, islot):
            pltpu.make_async_copy(
                mbuf[b], acc.at[idxb.at[islot, 2]], semS[b]).wait()

        def body(q, sub, do_wait_s, do_next, do_issue_i):
            b = sub % 2
            islot = sub % 4
            i2 = (sub + 2) % 4
            wait_g(b, islot)
            if do_wait_s:
                wait_s(b, i2)
            if do_issue_i:
                issue_i(q + sub + 2, i2)
            compute(b)
            issue_s(b, islot)
            if do_next:
                wait_i(q + sub + 2, i2)
                issue_g(q + sub + 2, b, i2)

        for k in range(4):
            issue_i(k, k)
        wait_i(0, 0)
        issue_g(0, 0, 0)
        wait_i(1, 1)
        issue_g(1, 1, 1)
        body(0, 0, False, True, False)
        body(0, 1, False, True, False)
        body(0, 2, True, True, True)
        body(0, 3, True, True, True)

        @pl.loop(4, Q - 4, step=4)
        def _(q):
            for sub in range(4):
                body(q, sub, True, True, True)

        body(Q - 4, 0, True, True, True)
        body(Q - 4, 1, True, True, True)
        body(Q - 4, 2, True, False, False)
        body(Q - 4, 3, True, False, False)
        wait_s(0, 2)
        wait_s(1, 3)

        plsc.subcore_barrier()

        r0 = s * rows_per_sub
        pltpu.sync_copy(acc.at[pl.ds(r0, rows_per_sub)],
                        out_hbm.at[c, pl.ds(r0, rows_per_sub)])

    return edge_kernel(tabAf, tabBf, eidx)


def kernel(x, adj_list_0, adj_list_1, adj_list_2, adj_list_3, W, b):
    adj = jnp.stack([adj_list_0, adj_list_1, adj_list_2, adj_list_3])
    srcs = jnp.concatenate(
        [adj[:, :, 0], jnp.zeros((T, E_PAD - E_PER), jnp.int32)], axis=1)
    tgts = jnp.concatenate(
        [adj[:, :, 1], jnp.full((T, E_PAD - E_PER), N, jnp.int32)], axis=1)

    offs = (jnp.arange(T, dtype=jnp.int32) * NROWS)[:, None]

    def _lay(a):
        return jnp.transpose(
            a.reshape(T, 16, CHUNKS, CH), (1, 0, 2, 3)).reshape(
                16, T * CHUNKS, CH)

    eidx = jnp.stack([_lay(srcs + offs), _lay(tgts + offs), _lay(tgts)],
                     axis=2)

    xp = jnp.pad(x, ((0, NROWS - N), (0, 0)))
    b8 = jnp.broadcast_to(b[:, None, :], (T, 8, MSG))

    tabA, tabB = _tables(xp, W, b8)
    out2 = _edge_stage(tabA.reshape(2, T * NROWS, HALF),
                       tabB.reshape(2, T * NROWS, HALF), eidx)
    return jnp.concatenate([out2[0, :N], out2[1, :N]], axis=1)
